# baseline (speedup 1.0000x reference)
import functools

import jax
import jax.numpy as jnp
from jax import lax
from jax.experimental import pallas as pl
from jax.experimental.pallas import tpu as pltpu
from jax.experimental.pallas import tpu_sc as plsc

MAX_L = 16
D = 256
B, N, E = 8, 4096, 8192


NC, NS = 2, 16
NW = NC * NS
ROWS = B * E
RW = ROWS // NW
R = 128
NCHUNK = RW // R


def _sc_body(x_hbm, idx_hbm, mask_hbm, tab_hbm, out_hbm,
             xbuf, idxbuf, maskbuf, tabbuf):
    wid = lax.axis_index("s") * NC + lax.axis_index("c")
    base = wid * RW

    pltpu.sync_copy(tab_hbm, tabbuf)
    pltpu.sync_copy(idx_hbm.at[pl.ds(base, RW)], idxbuf)
    pltpu.sync_copy(mask_hbm.at[pl.ds(base, RW)], maskbuf)

    lane = lax.iota(jnp.int32, 16)

    def chunk_body(ci, carry):
        row0 = base + ci * R
        pltpu.sync_copy(x_hbm.at[pl.ds(row0, R)], xbuf)

        def group_body(g, carry2):
            o_vec = idxbuf[pl.ds(ci * R + g * 16, 16)]
            m_vec = maskbuf[pl.ds(ci * R + g * 16, 16)]
            row_idx = g * 16 + lane

            def col_body(c, carry3):
                col_vec = jnp.full((16,), c, jnp.int32)
                xv = plsc.load_gather(xbuf, [row_idx, col_vec])
                bv = plsc.load_gather(tabbuf, [o_vec, col_vec])
                plsc.store_scatter(xbuf, [row_idx, col_vec],
                                   (xv + bv) * m_vec)
                return carry3

            return lax.fori_loop(0, D, col_body, carry2)

        lax.fori_loop(0, R // 16, group_body, carry)
        pltpu.sync_copy(xbuf, out_hbm.at[pl.ds(row0, R)])
        return carry

    lax.fori_loop(0, NCHUNK, chunk_body, jnp.int32(0))


_sc_call = functools.partial(
    pl.kernel,
    out_type=jax.ShapeDtypeStruct((ROWS, D), jnp.float32),
    mesh=plsc.VectorSubcoreMesh(core_axis_name="c", subcore_axis_name="s"),
    compiler_params=pltpu.CompilerParams(use_tc_tiling_on_sc=False,
                                         needs_layout_passes=False),
    scratch_types=[
        pltpu.VMEM((R, D), jnp.float32),
        pltpu.VMEM((RW,), jnp.int32),
        pltpu.VMEM((RW,), jnp.float32),
        pltpu.VMEM((MAX_L + 1, D), jnp.float32),
    ],
)(_sc_body)


VBLK = 512
VGRID = (B * N) // VBLK


def _tc_body(x_ref, m_ref, w_ref, o_ref):
    bias = w_ref[1:2, :]
    m = m_ref[...][0, 0][:, None]
    o_ref[...] = (x_ref[...] + bias[None]) * m[None]


_tc_call = pl.pallas_call(
    _tc_body,
    grid=(VGRID,),
    in_specs=[
        pl.BlockSpec((1, VBLK, D), lambda i: (i, 0, 0)),
        pl.BlockSpec((1, 1, VBLK), lambda i: (i, 0, 0)),
        pl.BlockSpec((MAX_L + 1, D), lambda i: (0, 0)),
    ],
    out_specs=pl.BlockSpec((1, VBLK, D), lambda i: (i, 0, 0)),
    out_shape=jax.ShapeDtypeStruct((VGRID, VBLK, D), jnp.float32),
)


def kernel(x_v, x_e, edge_orders, node_mask, edge_mask, b_weight):
    idx = edge_orders.astype(jnp.int32).reshape(ROWS)
    emask = edge_mask.astype(jnp.float32).reshape(ROWS)
    nmask = node_mask.astype(jnp.float32).reshape(VGRID, 1, VBLK)
    xe = x_e.reshape(ROWS, D)
    xv = x_v.reshape(VGRID, VBLK, D)

    out_e = _sc_call(xe, idx, emask, b_weight)
    out_v = _tc_call(xv, nmask, b_weight)
    return (out_v.reshape(B, N, D), out_e.reshape(B, E, D))


# speedup vs baseline: 2.9470x; 2.9470x over previous
import functools

import jax
import jax.numpy as jnp
from jax import lax
from jax.experimental import pallas as pl
from jax.experimental.pallas import tpu as pltpu
from jax.experimental.pallas import tpu_sc as plsc

MAX_L = 16
D = 256
B, N, E = 8, 4096, 8192


NC, NS = 2, 16
NW = NC * NS
ROWS = B * E
RW = ROWS // NW
R = 128
NCHUNK = RW // R


def _sc_body(x_hbm, idx_hbm, mask_hbm, tab_hbm, out_hbm,
             xbufs, idxbuf, maskbuf, tabbuf, in_sems, out_sems):
    wid = lax.axis_index("s") * NC + lax.axis_index("c")
    base = wid * RW

    pltpu.sync_copy(tab_hbm, tabbuf)
    pltpu.sync_copy(idx_hbm.at[pl.ds(base, RW)], idxbuf)
    pltpu.sync_copy(mask_hbm.at[pl.ds(base, RW)], maskbuf)

    lane = lax.iota(jnp.int32, 16)

    def start_in(ci, b):
        return pltpu.async_copy(
            x_hbm.at[pl.ds((base + ci * R) * D, R * D)], xbufs[b],
            in_sems[b])

    def start_out(ci, b):
        return pltpu.async_copy(
            xbufs[b], out_hbm.at[pl.ds((base + ci * R) * D, R * D)],
            out_sems[b])

    def compute(ci, b):
        xbuf = xbufs[b]

        def row_body(r, carry):
            wrow = jnp.full((16,), ci * R + r, jnp.int32)
            o_splat = plsc.load_gather(idxbuf, [wrow])
            m_splat = plsc.load_gather(maskbuf, [wrow])
            obase = o_splat * D + lane
            rbase = r * D
            for c in range(D // 16):
                sl = pl.ds(rbase + c * 16, 16)
                xv = xbuf[sl]
                bv = plsc.load_gather(tabbuf, [obase + (c * 16)])
                xbuf[sl] = (xv + bv) * m_splat
            return carry

        lax.fori_loop(0, R, row_body, jnp.int32(0))

    in_copy = [None, None]
    out_copy = [None, None]
    in_copy[0] = start_in(0, 0)
    for ci in range(NCHUNK):
        b = ci % 2
        in_copy[b].wait()
        if ci + 1 < NCHUNK:
            nb = (ci + 1) % 2
            if out_copy[nb] is not None:
                out_copy[nb].wait()
            in_copy[nb] = start_in(ci + 1, nb)
        compute(ci, b)
        out_copy[b] = start_out(ci, b)
    for oc in out_copy:
        if oc is not None:
            oc.wait()


_sc_call = functools.partial(
    pl.kernel,
    out_type=jax.ShapeDtypeStruct((ROWS * D,), jnp.float32),
    mesh=plsc.VectorSubcoreMesh(core_axis_name="c", subcore_axis_name="s"),
    compiler_params=pltpu.CompilerParams(use_tc_tiling_on_sc=False,
                                         needs_layout_passes=False),
    scratch_types=[
        [pltpu.VMEM((R * D,), jnp.float32) for _ in range(2)],
        pltpu.VMEM((RW,), jnp.int32),
        pltpu.VMEM((RW,), jnp.float32),
        pltpu.VMEM(((MAX_L + 1) * D,), jnp.float32),
        [pltpu.SemaphoreType.DMA for _ in range(2)],
        [pltpu.SemaphoreType.DMA for _ in range(2)],
    ],
)(_sc_body)


VBLK = 512
VGRID = (B * N) // VBLK


def _tc_body(x_ref, m_ref, w_ref, o_ref):
    bias = w_ref[1:2, :]
    m = m_ref[...][0, 0][:, None]
    o_ref[...] = (x_ref[...] + bias[None]) * m[None]


_tc_call = pl.pallas_call(
    _tc_body,
    grid=(VGRID,),
    in_specs=[
        pl.BlockSpec((1, VBLK, D), lambda i: (i, 0, 0)),
        pl.BlockSpec((1, 1, VBLK), lambda i: (i, 0, 0)),
        pl.BlockSpec((MAX_L + 1, D), lambda i: (0, 0)),
    ],
    out_specs=pl.BlockSpec((1, VBLK, D), lambda i: (i, 0, 0)),
    out_shape=jax.ShapeDtypeStruct((VGRID, VBLK, D), jnp.float32),
)


def kernel(x_v, x_e, edge_orders, node_mask, edge_mask, b_weight):
    idx = edge_orders.astype(jnp.int32).reshape(ROWS)
    emask = edge_mask.astype(jnp.float32).reshape(ROWS)
    nmask = node_mask.astype(jnp.float32).reshape(VGRID, 1, VBLK)
    xe = x_e.reshape(ROWS * D)
    xv = x_v.reshape(VGRID, VBLK, D)

    out_e = _sc_call(xe, idx, emask, b_weight.reshape((MAX_L + 1) * D))
    out_v = _tc_call(xv, nmask, b_weight)
    return (out_v.reshape(B, N, D), out_e.reshape(B, E, D))
